# baseline (speedup 1.0000x reference)
import functools

import jax
import jax.numpy as jnp
from jax import lax
from jax.experimental import pallas as pl
from jax.experimental.pallas import tpu as pltpu
from jax.experimental.pallas import tpu_sc as plsc

B = 16384
D = 64
H = 128
NC = 2
NS = 16
NW = NC * NS
BPW = B // NW
CH = BPW // 2


def _sc_gather_body(pair_idx, item_t, c1_t, c2_t, c3_t, e_out,
                    idx0, idx1, idx2, idx3, rows0, rows1, sem0, sem1):
    wid = lax.axis_index("s") * NC + lax.axis_index("c")
    base = wid * BPW

    idxs = (idx0, idx1, idx2, idx3)
    for t in range(4):
        pltpu.sync_copy(pair_idx.at[t, pl.ds(base, BPW)], idxs[t])

    tabs = (item_t, c1_t, c2_t, c3_t)
    bufs = (rows0, rows1)
    sems = (sem0, sem1)
    pending = [None, None]
    dst = [None, None]
    step = 0
    for t in range(4):
        for c in range(2):
            s = step % 2
            if pending[s] is not None:
                pending[s].wait()
                pltpu.sync_copy(bufs[s], e_out.at[dst[s][0], pl.ds(dst[s][1], CH)])
            pending[s] = pltpu.async_copy(
                tabs[t].at[idxs[t].at[pl.ds(c * CH, CH)]], bufs[s], sems[s])
            dst[s] = (t, base + c * CH)
            step += 1
    for s in range(2):
        pending[s].wait()
        pltpu.sync_copy(bufs[s], e_out.at[dst[s][0], pl.ds(dst[s][1], CH)])


@functools.cache
def _sc_gather():
    return pl.kernel(
        _sc_gather_body,
        out_type=jax.ShapeDtypeStruct((4, B, 2 * D), jnp.float32),
        mesh=plsc.VectorSubcoreMesh(core_axis_name="c", subcore_axis_name="s"),
        scratch_types=[
            pltpu.VMEM((BPW,), jnp.int32),
            pltpu.VMEM((BPW,), jnp.int32),
            pltpu.VMEM((BPW,), jnp.int32),
            pltpu.VMEM((BPW,), jnp.int32),
            pltpu.VMEM((CH, 2 * D), jnp.float32),
            pltpu.VMEM((CH, 2 * D), jnp.float32),
            pltpu.SemaphoreType.DMA,
            pltpu.SemaphoreType.DMA,
        ],
    )


def _mlp_body(e_ref, par_ref, w1_ref, b1_ref, w2_ref, b2_ref, out_ref):
    h = None
    for t in range(4):
        odd = (par_ref[t][:, None] == 1)
        xt = jnp.where(odd, e_ref[t, :, D:2 * D], e_ref[t, :, 0:D])
        ht = jnp.dot(xt, w1_ref[t * D:(t + 1) * D],
                     preferred_element_type=jnp.float32)
        h = ht if h is None else h + ht
    h = jnp.maximum(h + b1_ref[...], 0.0)
    out_ref[...] = jnp.dot(h, w2_ref[...], preferred_element_type=jnp.float32) + b2_ref[...]


def _mlp(e, par, w1, b1, w2, b2, blk=2048):
    grid = (B // blk,)
    return pl.pallas_call(
        _mlp_body,
        grid=grid,
        in_specs=[
            pl.BlockSpec((4, blk, 2 * D), lambda i: (0, i, 0)),
            pl.BlockSpec((4, blk), lambda i: (0, i)),
            pl.BlockSpec((4 * D, H), lambda i: (0, 0)),
            pl.BlockSpec((1, H), lambda i: (0, 0)),
            pl.BlockSpec((H, D), lambda i: (0, 0)),
            pl.BlockSpec((1, D), lambda i: (0, 0)),
        ],
        out_specs=pl.BlockSpec((blk, D), lambda i: (i, 0)),
        out_shape=jax.ShapeDtypeStruct((B, D), jnp.float32),
    )(e, par, w1, b1, w2, b2)


def kernel(item_id, category, category2, category3,
           item_table, cat1_table, cat2_table, cat3_table,
           W1, b1, W2, b2):
    idx = jnp.stack([item_id, category, category2, category3])
    pair_idx = idx >> 1
    par = idx & 1
    e = _sc_gather()(pair_idx,
                     item_table.reshape(-1, 2 * D),
                     cat1_table.reshape(-1, 2 * D),
                     cat2_table.reshape(-1, 2 * D),
                     cat3_table.reshape(-1, 2 * D))
    return _mlp(e, par, W1, b1.reshape(1, H), W2, b2.reshape(1, D))


# speedup vs baseline: 2.0766x; 2.0766x over previous
import functools

import jax
import jax.numpy as jnp
from jax import lax
from jax.experimental import pallas as pl
from jax.experimental.pallas import tpu as pltpu
from jax.experimental.pallas import tpu_sc as plsc

B = 16384
D = 64
H = 128
V = 1000000
Q = 524288
WB = 8192
NC = 2
NS = 16
NW = NC * NS
BPW = B // NW
CH = BPW // 2


def _tr_body(a_ref, b_ref, out_ref):
    a = a_ref[...]
    b = b_ref[...]
    out_ref[...] = jnp.concatenate([a.T, b.T], axis=1)


def _pair_table(tabT):
    nright = V // WB
    return pl.pallas_call(
        _tr_body,
        grid=(Q // WB,),
        in_specs=[
            pl.BlockSpec((D, WB), lambda i: (0, i)),
            pl.BlockSpec((D, WB), lambda i: (0, jnp.minimum(i + Q // WB, nright))),
        ],
        out_specs=pl.BlockSpec((WB, 128), lambda i: (i, 0)),
        out_shape=jax.ShapeDtypeStruct((Q, 128), jnp.float32),
    )(tabT, tabT)


def _sc_gather_body(pair_idx, item_t, c1_t, c2_t, c3_t, e_out,
                    idx0, idx1, idx2, idx3, rows0, rows1, sem0, sem1):
    wid = lax.axis_index("s") * NC + lax.axis_index("c")
    base = wid * BPW

    idxs = (idx0, idx1, idx2, idx3)
    for t in range(4):
        pltpu.sync_copy(pair_idx.at[t, pl.ds(base, BPW)], idxs[t])

    tabs = (item_t, c1_t, c2_t, c3_t)
    bufs = (rows0, rows1)
    sems = (sem0, sem1)
    pending = [None, None]
    dst = [None, None]
    step = 0
    for t in range(4):
        for c in range(2):
            s = step % 2
            if pending[s] is not None:
                pending[s].wait()
                pltpu.sync_copy(bufs[s], e_out.at[dst[s][0], pl.ds(dst[s][1], CH)])
            pending[s] = pltpu.async_copy(
                tabs[t].at[idxs[t].at[pl.ds(c * CH, CH)]], bufs[s], sems[s])
            dst[s] = (t, base + c * CH)
            step += 1
    for s in range(2):
        pending[s].wait()
        pltpu.sync_copy(bufs[s], e_out.at[dst[s][0], pl.ds(dst[s][1], CH)])


@functools.cache
def _sc_gather():
    return pl.kernel(
        _sc_gather_body,
        out_type=jax.ShapeDtypeStruct((4, B, 2 * D), jnp.float32),
        mesh=plsc.VectorSubcoreMesh(core_axis_name="c", subcore_axis_name="s"),
        scratch_types=[
            pltpu.VMEM((BPW,), jnp.int32),
            pltpu.VMEM((BPW,), jnp.int32),
            pltpu.VMEM((BPW,), jnp.int32),
            pltpu.VMEM((BPW,), jnp.int32),
            pltpu.VMEM((CH, 2 * D), jnp.float32),
            pltpu.VMEM((CH, 2 * D), jnp.float32),
            pltpu.SemaphoreType.DMA,
            pltpu.SemaphoreType.DMA,
        ],
    )


def _mlp_body(e_ref, par_ref, w1_ref, b1_ref, w2_ref, b2_ref, out_ref):
    h = None
    for t in range(4):
        odd = (par_ref[t][:, None] == 1)
        xt = jnp.where(odd, e_ref[t, :, D:2 * D], e_ref[t, :, 0:D])
        ht = jnp.dot(xt, w1_ref[t * D:(t + 1) * D],
                     preferred_element_type=jnp.float32)
        h = ht if h is None else h + ht
    h = jnp.maximum(h + b1_ref[...], 0.0)
    out_ref[...] = jnp.dot(h, w2_ref[...], preferred_element_type=jnp.float32) + b2_ref[...]


def _mlp(e, par, w1, b1, w2, b2, blk=2048):
    return pl.pallas_call(
        _mlp_body,
        grid=(B // blk,),
        in_specs=[
            pl.BlockSpec((4, blk, 2 * D), lambda i: (0, i, 0)),
            pl.BlockSpec((4, blk), lambda i: (0, i)),
            pl.BlockSpec((4 * D, H), lambda i: (0, 0)),
            pl.BlockSpec((1, H), lambda i: (0, 0)),
            pl.BlockSpec((H, D), lambda i: (0, 0)),
            pl.BlockSpec((1, D), lambda i: (0, 0)),
        ],
        out_specs=pl.BlockSpec((blk, D), lambda i: (i, 0)),
        out_shape=jax.ShapeDtypeStruct((B, D), jnp.float32),
    )(e, par, w1, b1, w2, b2)


def kernel(item_id, category, category2, category3,
           item_table, cat1_table, cat2_table, cat3_table,
           W1, b1, W2, b2):
    pairs_item = _pair_table(item_table.T)

    par_item = (item_id >= Q).astype(jnp.int32)
    pair_idx = jnp.stack([item_id - Q * par_item,
                          category >> 1, category2 >> 1, category3 >> 1])
    par = jnp.stack([par_item, category & 1, category2 & 1, category3 & 1])

    e = _sc_gather()(pair_idx, pairs_item,
                     cat1_table.reshape(-1, 2 * D),
                     cat2_table.reshape(-1, 2 * D),
                     cat3_table.reshape(-1, 2 * D))
    return _mlp(e, par, W1, b1.reshape(1, H), W2, b2.reshape(1, D))


# speedup vs baseline: 2.6853x; 1.2931x over previous
import functools

import jax
import jax.numpy as jnp
from jax import lax
from jax.experimental import pallas as pl
from jax.experimental.pallas import tpu as pltpu
from jax.experimental.pallas import tpu_sc as plsc

B = 16384
D = 64
H = 128
V = 1000000
QR = 524288
Q2 = 262144
WB = 4096
NC = 2
NS = 16
NW = NC * NS
BPW = B // NW
CH = BPW // 2


def _tr_body(a_ref, b_ref, out_ref):
    a16 = a_ref[...].T.astype(jnp.bfloat16)
    b16 = b_ref[...].T.astype(jnp.bfloat16)
    pa = pltpu.bitcast(a16, jnp.float32)
    pb = pltpu.bitcast(b16, jnp.float32)
    out_ref[...] = jnp.concatenate([pa, pb], axis=1)


def _pair_table(tabT):
    nlast = V // (2 * WB)
    return pl.pallas_call(
        _tr_body,
        grid=(Q2 // WB,),
        in_specs=[
            pl.BlockSpec((D, 2 * WB), lambda i: (0, i)),
            pl.BlockSpec((D, 2 * WB),
                         lambda i: (0, jnp.minimum(i + QR // (2 * WB), nlast))),
        ],
        out_specs=pl.BlockSpec((WB, 128), lambda i: (i, 0)),
        out_shape=jax.ShapeDtypeStruct((Q2, 128), jnp.float32),
    )(tabT, tabT)


def _sc_gather_body(pair_idx, item_t, c1_t, c2_t, c3_t, e_out,
                    idx0, idx1, idx2, idx3, rows0, rows1, sem0, sem1):
    wid = lax.axis_index("s") * NC + lax.axis_index("c")
    base = wid * BPW

    idxs = (idx0, idx1, idx2, idx3)
    for t in range(4):
        pltpu.sync_copy(pair_idx.at[t, pl.ds(base, BPW)], idxs[t])

    tabs = (item_t, c1_t, c2_t, c3_t)
    bufs = (rows0, rows1)
    sems = (sem0, sem1)
    pending = [None, None]
    dst = [None, None]
    step = 0
    for t in range(4):
        for c in range(2):
            s = step % 2
            if pending[s] is not None:
                pending[s].wait()
                pltpu.sync_copy(bufs[s], e_out.at[dst[s][0], pl.ds(dst[s][1], CH)])
            pending[s] = pltpu.async_copy(
                tabs[t].at[idxs[t].at[pl.ds(c * CH, CH)]], bufs[s], sems[s])
            dst[s] = (t, base + c * CH)
            step += 1
    for s in range(2):
        pending[s].wait()
        pltpu.sync_copy(bufs[s], e_out.at[dst[s][0], pl.ds(dst[s][1], CH)])


@functools.cache
def _sc_gather():
    return pl.kernel(
        _sc_gather_body,
        out_type=jax.ShapeDtypeStruct((4, B, 2 * D), jnp.float32),
        mesh=plsc.VectorSubcoreMesh(core_axis_name="c", subcore_axis_name="s"),
        scratch_types=[
            pltpu.VMEM((BPW,), jnp.int32),
            pltpu.VMEM((BPW,), jnp.int32),
            pltpu.VMEM((BPW,), jnp.int32),
            pltpu.VMEM((BPW,), jnp.int32),
            pltpu.VMEM((CH, 2 * D), jnp.float32),
            pltpu.VMEM((CH, 2 * D), jnp.float32),
            pltpu.SemaphoreType.DMA,
            pltpu.SemaphoreType.DMA,
        ],
    )


def _mlp_body(e_ref, par_ref, sub_ref, w1_ref, b1_ref, w2_ref, b2_ref, out_ref):
    h = None
    for t in range(4):
        odd = (par_ref[t][:, None] == 1)
        xt = jnp.where(odd, e_ref[t, :, D:2 * D], e_ref[t, :, 0:D])
        if t == 0:
            u = jax.lax.bitcast_convert_type(xt, jnp.int32)
            hi = (sub_ref[0][:, None] == 1)
            chosen = jnp.where(hi, u & jnp.int32(-65536), u << 16)
            xt = jax.lax.bitcast_convert_type(chosen, jnp.float32)
        ht = jnp.dot(xt, w1_ref[t * D:(t + 1) * D],
                     preferred_element_type=jnp.float32)
        h = ht if h is None else h + ht
    h = jnp.maximum(h + b1_ref[...], 0.0)
    out_ref[...] = jnp.dot(h, w2_ref[...], preferred_element_type=jnp.float32) + b2_ref[...]


def _mlp(e, par, sub, w1, b1, w2, b2, blk=2048):
    return pl.pallas_call(
        _mlp_body,
        grid=(B // blk,),
        in_specs=[
            pl.BlockSpec((4, blk, 2 * D), lambda i: (0, i, 0)),
            pl.BlockSpec((4, blk), lambda i: (0, i)),
            pl.BlockSpec((1, blk), lambda i: (0, i)),
            pl.BlockSpec((4 * D, H), lambda i: (0, 0)),
            pl.BlockSpec((1, H), lambda i: (0, 0)),
            pl.BlockSpec((H, D), lambda i: (0, 0)),
            pl.BlockSpec((1, D), lambda i: (0, 0)),
        ],
        out_specs=pl.BlockSpec((blk, D), lambda i: (i, 0)),
        out_shape=jax.ShapeDtypeStruct((B, D), jnp.float32),
    )(e, par, sub, w1, b1, w2, b2)


def kernel(item_id, category, category2, category3,
           item_table, cat1_table, cat2_table, cat3_table,
           W1, b1, W2, b2):
    pairs_item = _pair_table(item_table.T)

    reg = (item_id >= QR).astype(jnp.int32)
    loc = item_id - QR * reg
    pair_idx = jnp.stack([loc >> 1,
                          category >> 1, category2 >> 1, category3 >> 1])
    par = jnp.stack([reg, category & 1, category2 & 1, category3 & 1])
    sub = (loc & 1).reshape(1, B)

    e = _sc_gather()(pair_idx, pairs_item,
                     cat1_table.reshape(-1, 2 * D),
                     cat2_table.reshape(-1, 2 * D),
                     cat3_table.reshape(-1, 2 * D))
    return _mlp(e, par, sub, W1, b1.reshape(1, H), W2, b2.reshape(1, D))
